# baseline (speedup 1.0000x reference)
import functools

import jax
import jax.numpy as jnp
from jax import lax
from jax.experimental import pallas as pl
from jax.experimental.pallas import tpu as pltpu
from jax.experimental.pallas import tpu_sc as plsc


@functools.cache
def _make_gather(V, Ep, B, C):
    info = plsc.get_sparse_core_info()
    num_workers = info.num_cores * info.num_subcores
    b_per_w = B // num_workers
    n_chunks = b_per_w // C
    assert b_per_w % C == 0 and B % num_workers == 0

    mesh = plsc.VectorSubcoreMesh(core_axis_name="c", subcore_axis_name="s")

    @functools.partial(
        pl.kernel,
        mesh=mesh,
        out_type=jax.ShapeDtypeStruct((B, Ep), jnp.float32),
        scratch_types=[
            pltpu.VMEM((C,), jnp.int32),
            pltpu.VMEM((C, Ep), jnp.float32),
            pltpu.SemaphoreType.DMA,
        ],
    )
    def gather_kernel(table_hbm, idx_hbm, out_hbm, idx_v, rows_v, sem):
        wid = lax.axis_index("s") * info.num_cores + lax.axis_index("c")
        base = wid * b_per_w

        def body(i, carry):
            off = base + i * C
            pltpu.sync_copy(idx_hbm.at[pl.ds(off, C)], idx_v)
            pltpu.async_copy(table_hbm.at[idx_v], rows_v, sem).wait()
            pltpu.sync_copy(rows_v, out_hbm.at[pl.ds(off, C)])
            return carry

        lax.fori_loop(0, n_chunks, body, 0)

    return gather_kernel


def _head_body(emb_ref, w_ref, b_ref, out_ref):
    rb, t, ep = emb_ref.shape
    emb = emb_ref[...].reshape(rb * t, ep)
    acc = jnp.dot(emb, w_ref[...], preferred_element_type=jnp.float32) + b_ref[...]
    out_ref[...] = acc.reshape(rb, t, -1)


@functools.cache
def _make_head(Bb, T, Ep, Vo, RB):
    grid = (Bb // RB,)
    return pl.pallas_call(
        _head_body,
        grid=grid,
        in_specs=[
            pl.BlockSpec((RB, T, Ep), lambda i: (i, 0, 0)),
            pl.BlockSpec((Ep, Vo), lambda i: (0, 0)),
            pl.BlockSpec((1, Vo), lambda i: (0, 0)),
        ],
        out_specs=pl.BlockSpec((RB, T, Vo), lambda i: (i, 0, 0)),
        out_shape=jax.ShapeDtypeStruct((Bb, T, Vo), jnp.float32),
    )


def kernel(idx, tok_table, pos_table, W, b):
    Bb, T = idx.shape
    V, E = tok_table.shape
    Vo = W.shape[1]
    B = Bb * T
    Ep = 128

    tok_p = jnp.pad(tok_table, ((0, 0), (0, Ep - E)))
    W_p = jnp.pad(W, ((0, Ep - E), (0, 0)))
    flat_idx = idx.reshape(-1).astype(jnp.int32)

    emb = _make_gather(V, Ep, B, 512)(tok_p, flat_idx)
    emb3 = emb.reshape(Bb, T, Ep)
    return _make_head(Bb, T, Ep, Vo, 64)(emb3, W_p, b.reshape(1, Vo))


# speedup vs baseline: 1.1344x; 1.1344x over previous
import functools

import jax
import jax.numpy as jnp
from jax import lax
from jax.experimental import pallas as pl
from jax.experimental.pallas import tpu as pltpu
from jax.experimental.pallas import tpu_sc as plsc


@functools.cache
def _make_gather(V, Ep, B, C):
    info = plsc.get_sparse_core_info()
    num_workers = info.num_cores * info.num_subcores
    b_per_w = B // num_workers
    n_chunks = b_per_w // C
    assert b_per_w % C == 0 and B % num_workers == 0

    mesh = plsc.VectorSubcoreMesh(core_axis_name="c", subcore_axis_name="s")

    @functools.partial(
        pl.kernel,
        mesh=mesh,
        out_type=jax.ShapeDtypeStruct((B, Ep), jnp.float32),
        scratch_types=[
            pltpu.VMEM((C,), jnp.int32),
            pltpu.VMEM((C, Ep), jnp.float32),
            pltpu.SemaphoreType.DMA,
        ],
    )
    def gather_kernel(table_hbm, idx_hbm, out_hbm, idx_v, rows_v, sem):
        wid = lax.axis_index("s") * info.num_cores + lax.axis_index("c")
        base = wid * b_per_w

        def body(i, carry):
            off = base + i * C
            pltpu.sync_copy(idx_hbm.at[pl.ds(off, C)], idx_v)
            pltpu.async_copy(table_hbm.at[idx_v], rows_v, sem).wait()
            pltpu.sync_copy(rows_v, out_hbm.at[pl.ds(off, C)])
            return carry

        lax.fori_loop(0, n_chunks, body, 0)

    return gather_kernel


def _head_body(emb_ref, w_ref, b_ref, out_ref):
    rb, t, ep = emb_ref.shape
    emb = emb_ref[...].reshape(rb * t, ep)
    acc = jnp.dot(emb, w_ref[...], preferred_element_type=jnp.float32) + b_ref[...]
    out_ref[...] = acc.reshape(rb, t, -1)


@functools.cache
def _make_head(Bb, T, Ep, Vo, RB):
    grid = (Bb // RB,)
    return pl.pallas_call(
        _head_body,
        grid=grid,
        in_specs=[
            pl.BlockSpec((RB, T, Ep), lambda i: (i, 0, 0)),
            pl.BlockSpec((Ep, Vo), lambda i: (0, 0)),
            pl.BlockSpec((1, Vo), lambda i: (0, 0)),
        ],
        out_specs=pl.BlockSpec((RB, T, Vo), lambda i: (i, 0, 0)),
        out_shape=jax.ShapeDtypeStruct((Bb, T, Vo), jnp.float32),
    )


def kernel(idx, tok_table, pos_table, W, b):
    Bb, T = idx.shape
    V, E = tok_table.shape
    Vo = W.shape[1]
    B = Bb * T
    Ep = 128

    tok_p = jnp.pad(tok_table, ((0, 0), (0, Ep - E)))
    W_p = jnp.pad(W, ((0, Ep - E), (0, 0)))
    flat_idx = idx.reshape(-1).astype(jnp.int32)

    emb = _make_gather(V, Ep, B, 512)(tok_p, flat_idx)
    emb3 = emb.reshape(Bb, T, Ep)
    return _make_head(Bb, T, Ep, Vo, 512)(emb3, W_p, b.reshape(1, Vo))
